# baseline (speedup 1.0000x reference)
import jax
import jax.numpy as jnp
from jax.experimental import pallas as pl
from jax.experimental.pallas import tpu as pltpu

N_PRE = 5
N_CLS = 4
N_SUF = 68
N_TOK = 77
D = 512


def _body(lab_ref, sel_ref, tmpl_ref, cls_ref, out_ref):
    i = pl.program_id(0)
    sel = sel_ref[i]
    out_ref[...] = tmpl_ref[pl.ds(sel, 1)]
    out_ref[0, N_PRE:N_PRE + N_CLS, :] = cls_ref[0]


def kernel(label, cluster, cls_ctx, token_prefix, token_suffix,
           token_prefix_cluster, token_suffix_cluster):
    b = label.shape[0]
    sel = cluster.astype(jnp.int32)
    mid = jnp.zeros((1, N_CLS, D), dtype=token_prefix.dtype)
    tmpl = jnp.concatenate([
        jnp.concatenate([token_prefix, mid, token_suffix], axis=1),
        jnp.concatenate([token_prefix_cluster, mid, token_suffix_cluster], axis=1),
    ], axis=0)

    grid_spec = pltpu.PrefetchScalarGridSpec(
        num_scalar_prefetch=2,
        grid=(b,),
        in_specs=[
            pl.BlockSpec((2, N_TOK, D), lambda i, lab, sel: (0, 0, 0)),
            pl.BlockSpec((1, N_CLS, D), lambda i, lab, sel: (lab[i], 0, 0)),
        ],
        out_specs=pl.BlockSpec((1, N_TOK, D), lambda i, lab, sel: (i, 0, 0)),
    )
    out = pl.pallas_call(
        _body,
        grid_spec=grid_spec,
        out_shape=jax.ShapeDtypeStruct((b, N_TOK, D), cls_ctx.dtype),
    )(label, sel, tmpl, cls_ctx)
    return out
